# Initial kernel scaffold; baseline (speedup 1.0000x reference)
import functools
import math

import jax
import jax.numpy as jnp
from jax import lax
from jax.experimental import pallas as pl
from jax.experimental.pallas import tpu as pltpu
from jax.experimental.pallas import tpu_sc as plsc

NC = 2
NS = 16
L = 16
NW = NC * NS

B = 4096 * 200
D = 32
BPW = B // NW

IPS = 128
C = 1024
NR = C // IPS
NCHUNK = BPW // C

SCALE = math.sqrt(32.0)

_mesh = plsc.VectorSubcoreMesh(core_axis_name="c", subcore_axis_name="s")


@functools.partial(
    pl.kernel,
    out_type=jax.ShapeDtypeStruct((B, D), jnp.float32),
    mesh=_mesh,
    scratch_types=[
        pltpu.VMEM((NR, IPS), jnp.int32),
        pltpu.VMEM((C, D), jnp.float32),
        pltpu.SemaphoreType.DMA,
    ],
)
def _gather_scale(idx_hbm, table_hbm, out_hbm, idx_v, rows_v, sem):
    wid = lax.axis_index("s") * NC + lax.axis_index("c")
    base = wid * BPW

    def chunk_body(i, _):
        off = base + i * C
        pltpu.sync_copy(idx_hbm.at[pl.ds(off, C)], idx_v)
        copies = [
            pltpu.async_copy(
                table_hbm.at[idx_v.at[r]],
                rows_v.at[pl.ds(r * IPS, IPS)],
                sem,
            )
            for r in range(NR)
        ]
        for cp in copies:
            cp.wait()

        def scale_body(j, _):
            rows_v[j, pl.ds(0, L)] = rows_v[j, pl.ds(0, L)] * SCALE
            rows_v[j, pl.ds(L, L)] = rows_v[j, pl.ds(L, L)] * SCALE
            return 0

        lax.fori_loop(0, C, scale_body, 0, unroll=8)
        pltpu.sync_copy(rows_v, out_hbm.at[pl.ds(off, C)])
        return 0

    lax.fori_loop(0, NCHUNK, chunk_body, 0)


def kernel(x, table):
    flat_idx = x.reshape(B)
    out = _gather_scale(flat_idx, table)
    return out.reshape(x.shape[0], x.shape[1], D)


# speedup vs baseline: 1.3983x; 1.3983x over previous
import functools
import math

import jax
import jax.numpy as jnp
from jax import lax
from jax.experimental import pallas as pl
from jax.experimental.pallas import tpu as pltpu
from jax.experimental.pallas import tpu_sc as plsc

NC = 2
NS = 16
L = 16
NW = NC * NS

B = 4096 * 200
D = 32
BPW = B // NW

IPS = 128
C = 1024
NR = C // IPS
NCHUNK = BPW // C

SCALE = math.sqrt(32.0)

_mesh = plsc.VectorSubcoreMesh(core_axis_name="c", subcore_axis_name="s")


@functools.partial(
    pl.kernel,
    out_type=jax.ShapeDtypeStruct((B, D), jnp.float32),
    mesh=_mesh,
    scratch_types=[
        pltpu.VMEM((NR, IPS), jnp.int32),
        pltpu.VMEM((C, D), jnp.float32),
        pltpu.SemaphoreType.DMA,
    ],
    compiler_params=pltpu.CompilerParams(use_tc_tiling_on_sc=False),
)
def _gather_scale(idx_hbm, table_hbm, out_hbm, idx_v, rows_v, sem):
    wid = lax.axis_index("s") * NC + lax.axis_index("c")
    base = wid * BPW
    base_row = wid * (BPW // IPS)

    def chunk_body(i, _):
        off = base + i * C
        pltpu.sync_copy(idx_hbm.at[pl.ds(base_row + i * NR, NR)], idx_v)
        copies = [
            pltpu.async_copy(
                table_hbm.at[idx_v.at[r]],
                rows_v.at[pl.ds(r * IPS, IPS)],
                sem,
            )
            for r in range(NR)
        ]
        for cp in copies:
            cp.wait()

        def scale_body(j, _):
            rows_v[j, pl.ds(0, L)] = rows_v[j, pl.ds(0, L)] * SCALE
            rows_v[j, pl.ds(L, L)] = rows_v[j, pl.ds(L, L)] * SCALE
            return 0

        lax.fori_loop(0, C, scale_body, 0, unroll=8)
        pltpu.sync_copy(rows_v, out_hbm.at[pl.ds(off, C)])
        return 0

    lax.fori_loop(0, NCHUNK, chunk_body, 0)


def kernel(x, table):
    idx2d = x.reshape(B // IPS, IPS)
    out = _gather_scale(idx2d, table)
    return out.reshape(x.shape[0], x.shape[1], D)


# speedup vs baseline: 1.4762x; 1.0557x over previous
import functools
import math

import jax
import jax.numpy as jnp
from jax import lax
from jax.experimental import pallas as pl
from jax.experimental.pallas import tpu as pltpu
from jax.experimental.pallas import tpu_sc as plsc

NC = 2
NS = 16
L = 16
NW = NC * NS

B = 4096 * 200
D = 32
BPW = B // NW

IPS = 128
C = 640
NR = C // IPS
NCHUNK = BPW // C
IROWS = BPW // IPS
NBUF = 4
FA = 2

SCALE = math.sqrt(32.0)

_mesh = plsc.VectorSubcoreMesh(core_axis_name="c", subcore_axis_name="s")


@functools.partial(
    pl.kernel,
    out_type=jax.ShapeDtypeStruct((B, D), jnp.float32),
    mesh=_mesh,
    scratch_types=(
        [pltpu.VMEM((IROWS, IPS), jnp.int32)]
        + [pltpu.VMEM((C, D), jnp.float32) for _ in range(NBUF)]
        + [pltpu.SemaphoreType.DMA for _ in range(2 * NBUF)]
    ),
    compiler_params=pltpu.CompilerParams(use_tc_tiling_on_sc=False),
)
def _gather_scale(idx_hbm, table_hbm, out_hbm, idx_v, *bufs_and_sems):
    rows = bufs_and_sems[:NBUF]
    gsem = bufs_and_sems[NBUF:2 * NBUF]
    osem = bufs_and_sems[2 * NBUF:]

    wid = lax.axis_index("s") * NC + lax.axis_index("c")
    base = wid * BPW

    pltpu.sync_copy(idx_hbm.at[pl.ds(wid * IROWS, IROWS)], idx_v)

    def fire_gather(i):
        b = i % NBUF
        return [
            pltpu.async_copy(
                table_hbm.at[idx_v.at[i * NR + r]],
                rows[b].at[pl.ds(r * IPS, IPS)],
                gsem[b],
            )
            for r in range(NR)
        ]

    def scale_chunk(rv):
        def body(j, _):
            rv[j, pl.ds(0, L)] = rv[j, pl.ds(0, L)] * SCALE
            rv[j, pl.ds(L, L)] = rv[j, pl.ds(L, L)] * SCALE
            return 0

        lax.fori_loop(0, C, body, 0, unroll=8)

    gdesc = [None] * NCHUNK
    odesc = [None] * NCHUNK
    for i in range(FA):
        gdesc[i] = fire_gather(i)
    for i in range(NCHUNK):
        b = i % NBUF
        f = i + FA
        if f < NCHUNK:
            if f - NBUF >= 0:
                odesc[f - NBUF].wait()
            gdesc[f] = fire_gather(f)
        for cp in gdesc[i]:
            cp.wait()
        scale_chunk(rows[b])
        odesc[i] = pltpu.async_copy(
            rows[b], out_hbm.at[pl.ds(base + i * C, C)], osem[b]
        )
    for i in range(NCHUNK - NBUF, NCHUNK):
        odesc[i].wait()


def kernel(x, table):
    idx2d = x.reshape(B // IPS, IPS)
    out = _gather_scale(idx2d, table)
    return out.reshape(x.shape[0], x.shape[1], D)


# speedup vs baseline: 1.4767x; 1.0003x over previous
import functools
import math

import jax
import jax.numpy as jnp
from jax import lax
from jax.experimental import pallas as pl
from jax.experimental.pallas import tpu as pltpu
from jax.experimental.pallas import tpu_sc as plsc

NC = 2
NS = 16
L = 16
NW = NC * NS

B = 4096 * 200
D = 32
BPW = B // NW

IPS = 128
C = 640
NR = C // IPS
NCHUNK = BPW // C
IROWS = BPW // IPS
NBUF = 4
FA = 2

SCALE = math.sqrt(32.0)

_mesh = plsc.VectorSubcoreMesh(core_axis_name="c", subcore_axis_name="s")


@functools.partial(
    pl.kernel,
    out_type=jax.ShapeDtypeStruct((B, D), jnp.float32),
    mesh=_mesh,
    scratch_types=(
        [pltpu.VMEM((IROWS, IPS), jnp.int32)]
        + [pltpu.VMEM((C, D), jnp.float32) for _ in range(NBUF)]
        + [pltpu.SemaphoreType.DMA for _ in range(2 * NBUF)]
    ),
    compiler_params=pltpu.CompilerParams(use_tc_tiling_on_sc=False),
)
def _gather_scale(idx_hbm, table_hbm, out_hbm, idx_v, *bufs_and_sems):
    rows = bufs_and_sems[:NBUF]
    gsem = bufs_and_sems[NBUF:2 * NBUF]
    osem = bufs_and_sems[2 * NBUF:]

    wid = lax.axis_index("s") * NC + lax.axis_index("c")
    base = wid * BPW

    pltpu.sync_copy(idx_hbm.at[pl.ds(wid * IROWS, IROWS)], idx_v)

    def fire_gather(i):
        b = i % NBUF
        return [
            pltpu.async_copy(
                table_hbm.at[idx_v.at[i * NR + r]],
                rows[b].at[pl.ds(r * IPS, IPS)],
                gsem[b],
            )
            for r in range(NR)
        ]

    def scale_chunk(rv):
        @plsc.parallel_loop(0, C, step=1, unroll=8)
        def _(j):
            rv[j, pl.ds(0, L)] = rv[j, pl.ds(0, L)] * SCALE
            rv[j, pl.ds(L, L)] = rv[j, pl.ds(L, L)] * SCALE

    gdesc = [None] * NCHUNK
    odesc = [None] * NCHUNK
    for i in range(FA):
        gdesc[i] = fire_gather(i)
    for i in range(NCHUNK):
        b = i % NBUF
        f = i + FA
        if f < NCHUNK:
            if f - NBUF >= 0:
                odesc[f - NBUF].wait()
            gdesc[f] = fire_gather(f)
        for cp in gdesc[i]:
            cp.wait()
        scale_chunk(rows[b])
        odesc[i] = pltpu.async_copy(
            rows[b], out_hbm.at[pl.ds(base + i * C, C)], osem[b]
        )
    for i in range(NCHUNK - NBUF, NCHUNK):
        odesc[i].wait()


def kernel(x, table):
    idx2d = x.reshape(B // IPS, IPS)
    out = _gather_scale(idx2d, table)
    return out.reshape(x.shape[0], x.shape[1], D)
